# Initial kernel scaffold; baseline (speedup 1.0000x reference)
import jax
import jax.numpy as jnp
from jax.experimental import pallas as pl


def kernel(x, edge_index, edge_weight, W, b, gamma, beta, edge_enc):
    raise NotImplementedError("write your pallas kernel here")


# speedup vs baseline: 7.2605x; 7.2605x over previous
import functools

import jax
import jax.numpy as jnp
from jax import lax
from jax.experimental import pallas as pl
from jax.experimental.pallas import tpu as pltpu
from jax.experimental.pallas import tpu_sc as plsc

N = 10000
NP = 10240
D = 256
DH = 128
E = 160000
EF = 170240
PAD_NODE = 10200
L = 16
NSUB = 16
NW = 32
EPT = EF // NSUB
G = 112
NB = EPT // G
NPT = NP // NSUB

_mesh = plsc.VectorSubcoreMesh(core_axis_name="c", subcore_axis_name="s")


def _mm_body(x_ref, w_ref, b_ref, d_ref, o_ref):
    h = jnp.dot(x_ref[...], w_ref[...], preferred_element_type=jnp.float32)
    o_ref[...] = (h + b_ref[0:1, :]) * lax.rsqrt(d_ref[...])


def _linear(x, w_eff, b_eff, deg2d):
    bm = 1000
    return pl.pallas_call(
        _mm_body,
        grid=(2, N // bm),
        in_specs=[
            pl.BlockSpec((bm, D), lambda j, i: (i, 0)),
            pl.BlockSpec((D, DH), lambda j, i: (j, 0)),
            pl.BlockSpec((8, DH), lambda j, i: (j, 0)),
            pl.BlockSpec((bm, 1), lambda j, i: (i, 0)),
        ],
        out_specs=pl.BlockSpec((bm, DH), lambda j, i: (j * (N // bm) + i, 0)),
        out_shape=jax.ShapeDtypeStruct((2 * N, DH), jnp.float32),
    )(x, w_eff, b_eff, deg2d)


@functools.partial(
    pl.kernel,
    out_type=jax.ShapeDtypeStruct((NP,), jnp.float32),
    mesh=_mesh,
    scratch_types=[
        pltpu.VMEM((G,), jnp.int32),
        pltpu.VMEM((G, DH), jnp.float32),
        pltpu.VMEM((64, DH), jnp.float32),
        pltpu.VMEM((NPT,), jnp.float32),
        pltpu.VMEM_SHARED((NP, DH), jnp.float32),
    ],
)
def _deg_kernel(col_hbm, deg_hbm, cidx, ones, wbuf, dbuf, hist):
    c = lax.axis_index("c")
    s = lax.axis_index("s")
    z16 = jnp.zeros((L,), jnp.float32)
    o16 = jnp.ones((L,), jnp.float32)
    it16 = lax.iota(jnp.int32, L)

    def _zw(r, _):
        for j in range(DH // L):
            wbuf[r, pl.ds(j * L, L)] = z16
        return 0
    lax.fori_loop(0, 64, _zw, 0)
    for t in range(NPT // 64):
        pltpu.sync_copy(wbuf, hist.at[pl.ds(s * NPT + t * 64, 64)])

    def _o(i, _):
        for j in range(DH // L):
            ones[i, pl.ds(j * L, L)] = o16
        return 0
    lax.fori_loop(0, G, _o, 0)

    plsc.subcore_barrier()

    @pl.when(c == 0)
    def _scatter():
        def _chunk(k, _):
            pltpu.sync_copy(col_hbm.at[pl.ds(s * EPT + k * G, G)], cidx)
            pltpu.sync_copy(ones, hist.at[cidx], add=True)
            return 0
        lax.fori_loop(0, NB, _chunk, 0)

    plsc.subcore_barrier()

    @pl.when(c == 0)
    def _extract():
        for t in range(NPT // 64):
            pltpu.sync_copy(hist.at[pl.ds(s * NPT + t * 64, 64)], wbuf)

            def _grp(g, _, t=t):
                v = z16
                for l in range(L):
                    x = wbuf[g * L + l, pl.ds(0, L)][0]
                    v = jnp.where(it16 == l,
                                  jnp.full((L,), x, jnp.float32), v)
                dbuf[pl.ds(t * 64 + g * L, L)] = v
                return 0
            lax.fori_loop(0, 64 // L, _grp, 0)
        pltpu.sync_copy(dbuf, deg_hbm.at[pl.ds(s * NPT, NPT)])


def _rsqrt16(d):
    y = jnp.full((L,), 2.0 ** -10, jnp.float32)
    for thresh, seed in ((65536.0, 2.0 ** -8), (4096.0, 2.0 ** -6),
                         (256.0, 2.0 ** -4), (16.0, 2.0 ** -2)):
        y = jnp.where(d < thresh, jnp.full((L,), seed, jnp.float32), y)
    for _ in range(8):
        y = y * (1.5 - 0.5 * d * y * y)
    return y


@functools.partial(
    pl.kernel,
    out_type=jax.ShapeDtypeStruct((N, D), jnp.float32),
    mesh=_mesh,
    scratch_types=[
        pltpu.VMEM((G,), jnp.int32),
        pltpu.VMEM((G,), jnp.int32),
        pltpu.VMEM((G + L,), jnp.float32),
        pltpu.VMEM((G, DH), jnp.float32),
        pltpu.VMEM((G, DH), jnp.float32),
        pltpu.VMEM((64, DH), jnp.float32),
        pltpu.VMEM((NPT + L,), jnp.float32),
        pltpu.VMEM_SHARED((NP, DH), jnp.float32),
        pltpu.SemaphoreType.DMA,
    ],
)
def _agg_kernel(row_hbm, col_hbm, ew_hbm, hcat_hbm, deg_hbm, out_hbm,
                ridx, cidx, ewb, gbuf, sbuf, wbuf, disv, acc, sem):
    c = lax.axis_index("c")
    s = lax.axis_index("s")
    z16 = jnp.zeros((L,), jnp.float32)

    def _zw(r, _):
        for j in range(DH // L):
            wbuf[r, pl.ds(j * L, L)] = z16
        return 0
    lax.fori_loop(0, 64, _zw, 0)
    for t in range(NPT // 64):
        pltpu.sync_copy(wbuf, acc.at[pl.ds(s * NPT + t * 64, 64)])

    plsc.subcore_barrier()

    ebase = s * EPT
    coff = c * N

    def _batch(g, _):
        b = ebase + g * G
        pltpu.sync_copy(row_hbm.at[pl.ds(b, G)], ridx)
        pltpu.sync_copy(col_hbm.at[pl.ds(b, G)], cidx)
        pltpu.sync_copy(ew_hbm.at[pl.ds(b, G)], ewb.at[pl.ds(0, G)])

        def _adj(k, _):
            sl = pl.ds(k * L, L)
            ridx[sl] = ridx[sl] + coff
            return 0
        lax.fori_loop(0, G // L, _adj, 0)

        pltpu.async_copy(hcat_hbm.at[ridx], gbuf, sem).wait()

        def _edge(i, _):
            cf = jnp.full((L,), ewb[pl.ds(i, L)][0], jnp.float32)
            for j in range(DH // L):
                sl = pl.ds(j * L, L)
                sbuf[i, sl] = gbuf[i, sl] * cf
            return 0
        lax.fori_loop(0, G, _edge, 0)

        pltpu.sync_copy(sbuf, acc.at[cidx], add=True)
        return 0
    lax.fori_loop(0, NB, _batch, 0)

    plsc.subcore_barrier()

    pltpu.sync_copy(deg_hbm.at[pl.ds(s * NPT, NPT)], disv.at[pl.ds(0, NPT)])

    def _dis(k, _):
        sl = pl.ds(k * L, L)
        disv[sl] = _rsqrt16(disv[sl])
        return 0
    lax.fori_loop(0, NPT // L, _dis, 0)

    def _emit(t):
        rbase = s * NPT + t * 64
        pltpu.sync_copy(acc.at[pl.ds(rbase, 64)], wbuf)

        def _row(r, _):
            dv = jnp.full((L,), disv[pl.ds(t * 64 + r, L)][0], jnp.float32)
            for j in range(DH // L):
                sl = pl.ds(j * L, L)
                wbuf[r, sl] = wbuf[r, sl] * dv
            return 0
        lax.fori_loop(0, 64, _row, 0)

    @pl.when(s < NSUB - 1)
    def _full():
        for t in range(NPT // 64):
            _emit(t)
            pltpu.sync_copy(
                wbuf,
                out_hbm.at[pl.ds(s * NPT + t * 64, 64), pl.ds(c * DH, DH)])

    @pl.when(s == NSUB - 1)
    def _last():
        for t in range(6):
            _emit(t)
            pltpu.sync_copy(
                wbuf,
                out_hbm.at[pl.ds(s * NPT + t * 64, 64), pl.ds(c * DH, DH)])
        _emit(6)
        pltpu.sync_copy(
            wbuf.at[pl.ds(0, 16)],
            out_hbm.at[pl.ds(s * NPT + 6 * 64, 16), pl.ds(c * DH, DH)])


def kernel(x, edge_index, edge_weight, W, b, gamma, beta, edge_enc):
    ei = edge_index.astype(jnp.int32)
    scale = gamma * (1.0 / jnp.sqrt(jnp.float32(1.0 + 1e-5)))
    w_eff = (W * scale[:, None]).T.reshape(D, 2, DH).swapaxes(0, 1)
    w_eff = w_eff.reshape(2 * D, DH)
    b_eff = jnp.tile((b * scale + beta).reshape(2, 1, DH), (1, 8, 1))
    b_eff = b_eff.reshape(16, DH)

    npad = EF - E - N
    loop = jnp.arange(N, dtype=jnp.int32)
    row_full = jnp.concatenate(
        [ei[0], loop, jnp.zeros((npad,), jnp.int32)])
    col_full = jnp.concatenate(
        [ei[1], loop, jnp.full((npad,), PAD_NODE, jnp.int32)])
    ew_full = jnp.concatenate(
        [edge_weight * edge_enc.reshape(()), jnp.ones((N,), jnp.float32),
         jnp.zeros((npad,), jnp.float32)])

    deg = _deg_kernel(col_full)
    deg2d = deg[:N].reshape(N, 1)
    hcat = _linear(x, w_eff, b_eff, deg2d)
    out = _agg_kernel(row_full, col_full, ew_full, hcat, deg)
    return out


# speedup vs baseline: 9.3266x; 1.2846x over previous
import functools

import jax
import jax.numpy as jnp
from jax import lax
from jax.experimental import pallas as pl
from jax.experimental.pallas import tpu as pltpu
from jax.experimental.pallas import tpu_sc as plsc

N = 10000
NP = 10240
D = 256
DH = 128
E = 160000
EF = 170240
PAD_NODE = 10200
L = 16
NSUB = 16
NW = 32
EPT = EF // NSUB
G = 80
NB = EPT // G
EPTA = EF // NW
GA = 56
NBA = EPTA // GA
NPT = NP // NSUB

_mesh = plsc.VectorSubcoreMesh(core_axis_name="c", subcore_axis_name="s")


def _mm_body(x_ref, w_ref, b_ref, d_ref, o_ref):
    h = jnp.dot(x_ref[...], w_ref[...], preferred_element_type=jnp.float32)
    deg = d_ref[:, 0:1] + d_ref[:, 1:2]
    o_ref[...] = (h + b_ref[0:1, :]) * lax.rsqrt(deg)


def _linear(x, w_eff, b_eff, deg2d):
    bm = 1000
    return pl.pallas_call(
        _mm_body,
        grid=(2, N // bm),
        in_specs=[
            pl.BlockSpec((bm, D), lambda j, i: (i, 0)),
            pl.BlockSpec((D, DH), lambda j, i: (j, 0)),
            pl.BlockSpec((8, DH), lambda j, i: (j, 0)),
            pl.BlockSpec((bm, 2), lambda j, i: (i, 0)),
        ],
        out_specs=pl.BlockSpec((bm, DH), lambda j, i: (j * (N // bm) + i, 0)),
        out_shape=jax.ShapeDtypeStruct((2 * N, DH), jnp.float32),
    )(x, w_eff, b_eff, deg2d)


@functools.partial(
    pl.kernel,
    out_type=jax.ShapeDtypeStruct((2, NP), jnp.float32),
    mesh=_mesh,
    scratch_types=[
        pltpu.VMEM((GA,), jnp.int32),
        pltpu.VMEM((GA,), jnp.int32),
        pltpu.VMEM((GA, DH), jnp.float32),
        pltpu.VMEM((64, DH), jnp.float32),
        pltpu.VMEM((NPT,), jnp.float32),
        pltpu.VMEM_SHARED((NP, DH), jnp.float32),
        pltpu.SemaphoreType.DMA,
        pltpu.SemaphoreType.DMA,
    ],
)
def _deg_kernel(col_hbm, deg_hbm, cidx0, cidx1, ones, wbuf, dbuf, hist,
                sem0, sem1):
    c = lax.axis_index("c")
    s = lax.axis_index("s")
    z16 = jnp.zeros((L,), jnp.float32)
    o16 = jnp.ones((L,), jnp.float32)
    it16 = lax.iota(jnp.int32, L)

    def _zw(r, _):
        for j in range(DH // L):
            wbuf[r, pl.ds(j * L, L)] = z16
        return 0
    lax.fori_loop(0, 64, _zw, 0)
    for t in range(NPT // 64):
        pltpu.sync_copy(wbuf, hist.at[pl.ds(s * NPT + t * 64, 64)])

    def _o(i, _):
        for j in range(DH // L):
            ones[i, pl.ds(j * L, L)] = o16
        return 0
    lax.fori_loop(0, GA, _o, 0)

    plsc.subcore_barrier()

    ebase = (c * NSUB + s) * EPTA

    def _ld(k, cref, sem):
        return pltpu.async_copy(col_hbm.at[pl.ds(ebase + k * GA, GA)],
                                cref, sem)

    _ld(0, cidx0, sem0)

    def _pair(p, _):
        _ld(2 * p + 1, cidx1, sem1)
        pltpu.make_async_copy(col_hbm.at[pl.ds(0, GA)], cidx0, sem0).wait()
        pltpu.sync_copy(ones, hist.at[cidx0], add=True)
        _ld(2 * p + 2, cidx0, sem0)
        pltpu.make_async_copy(col_hbm.at[pl.ds(0, GA)], cidx1, sem1).wait()
        pltpu.sync_copy(ones, hist.at[cidx1], add=True)
        return 0
    lax.fori_loop(0, (NBA - 1) // 2, _pair, 0)
    pltpu.make_async_copy(col_hbm.at[pl.ds(0, GA)], cidx0, sem0).wait()
    pltpu.sync_copy(ones, hist.at[cidx0], add=True)

    plsc.subcore_barrier()

    for t in range(NPT // 64):
        pltpu.sync_copy(hist.at[pl.ds(s * NPT + t * 64, 64)], wbuf)

        def _grp(g, _, t=t):
            v = z16
            for l in range(L):
                x = wbuf[g * L + l, pl.ds(0, L)][0]
                v = jnp.where(it16 == l,
                              jnp.full((L,), x, jnp.float32), v)
            dbuf[pl.ds(t * 64 + g * L, L)] = v
            return 0
        lax.fori_loop(0, 64 // L, _grp, 0)
    pltpu.sync_copy(dbuf, deg_hbm.at[c, pl.ds(s * NPT, NPT)])


def _rsqrt16(d):
    y = jnp.full((L,), 2.0 ** -10, jnp.float32)
    for thresh, seed in ((65536.0, 2.0 ** -8), (4096.0, 2.0 ** -6),
                         (256.0, 2.0 ** -4), (16.0, 2.0 ** -2)):
        y = jnp.where(d < thresh, jnp.full((L,), seed, jnp.float32), y)
    for _ in range(8):
        y = y * (1.5 - 0.5 * d * y * y)
    return y


@functools.partial(
    pl.kernel,
    out_type=jax.ShapeDtypeStruct((N, D), jnp.float32),
    mesh=_mesh,
    scratch_types=[
        [pltpu.VMEM((G,), jnp.int32) for _ in range(2)],
        [pltpu.VMEM((G,), jnp.int32) for _ in range(2)],
        [pltpu.VMEM((G + L,), jnp.float32) for _ in range(2)],
        [pltpu.VMEM((G, DH), jnp.float32) for _ in range(2)],
        pltpu.VMEM((G, DH), jnp.float32),
        pltpu.VMEM((64, DH), jnp.float32),
        pltpu.VMEM((NPT + L,), jnp.float32),
        pltpu.VMEM((NPT,), jnp.float32),
        pltpu.VMEM_SHARED((NP, DH), jnp.float32),
        [pltpu.SemaphoreType.DMA for _ in range(2)],
    ],
)
def _agg_kernel(row_hbm, col_hbm, ew_hbm, hcat_hbm, deg_hbm, out_hbm,
                ridx, cidx, ewb, gbuf, sbuf, wbuf, disv, dtmp, acc, sem):
    c = lax.axis_index("c")
    s = lax.axis_index("s")
    z16 = jnp.zeros((L,), jnp.float32)

    def _zw(r, _):
        for j in range(DH // L):
            wbuf[r, pl.ds(j * L, L)] = z16
        return 0
    lax.fori_loop(0, 64, _zw, 0)
    for t in range(NPT // 64):
        pltpu.sync_copy(wbuf, acc.at[pl.ds(s * NPT + t * 64, 64)])

    plsc.subcore_barrier()

    ebase = s * EPT
    coff = c * N

    def _prep(g, q):
        b = ebase + g * G
        pltpu.sync_copy(row_hbm.at[pl.ds(b, G)], ridx[q])
        pltpu.sync_copy(col_hbm.at[pl.ds(b, G)], cidx[q])
        pltpu.sync_copy(ew_hbm.at[pl.ds(b, G)], ewb[q].at[pl.ds(0, G)])

        def _adj(k, _):
            sl = pl.ds(k * L, L)
            ridx[q][sl] = ridx[q][sl] + coff
            return 0
        lax.fori_loop(0, G // L, _adj, 0)
        pltpu.async_copy(hcat_hbm.at[ridx[q]], gbuf[q], sem[q])

    def _proc(q):
        pltpu.make_async_copy(hcat_hbm.at[ridx[q]], gbuf[q], sem[q]).wait()

        def _edge(i, _):
            cf = jnp.full((L,), ewb[q][pl.ds(i, L)][0], jnp.float32)
            for j in range(DH // L):
                sl = pl.ds(j * L, L)
                sbuf[i, sl] = gbuf[q][i, sl] * cf
            return 0
        lax.fori_loop(0, G, _edge, 0)
        pltpu.sync_copy(sbuf, acc.at[cidx[q]], add=True)

    _prep(0, 0)

    def _pair(p, _):
        _prep(2 * p + 1, 1)
        _proc(0)
        _prep(2 * p + 2, 0)
        _proc(1)
        return 0
    lax.fori_loop(0, (NB - 1) // 2, _pair, 0)
    _proc(0)

    plsc.subcore_barrier()

    pltpu.sync_copy(deg_hbm.at[0, pl.ds(s * NPT, NPT)],
                    disv.at[pl.ds(0, NPT)])
    pltpu.sync_copy(deg_hbm.at[1, pl.ds(s * NPT, NPT)], dtmp)

    def _dis(k, _):
        sl = pl.ds(k * L, L)
        disv[sl] = _rsqrt16(disv[sl] + dtmp[sl])
        return 0
    lax.fori_loop(0, NPT // L, _dis, 0)

    def _emit(t):
        rbase = s * NPT + t * 64
        pltpu.sync_copy(acc.at[pl.ds(rbase, 64)], wbuf)

        def _row(r, _):
            dv = jnp.full((L,), disv[pl.ds(t * 64 + r, L)][0], jnp.float32)
            for j in range(DH // L):
                sl = pl.ds(j * L, L)
                wbuf[r, sl] = wbuf[r, sl] * dv
            return 0
        lax.fori_loop(0, 64, _row, 0)

    @pl.when(s < NSUB - 1)
    def _full():
        for t in range(NPT // 64):
            _emit(t)
            pltpu.sync_copy(
                wbuf,
                out_hbm.at[pl.ds(s * NPT + t * 64, 64), pl.ds(c * DH, DH)])

    @pl.when(s == NSUB - 1)
    def _last():
        for t in range(6):
            _emit(t)
            pltpu.sync_copy(
                wbuf,
                out_hbm.at[pl.ds(s * NPT + t * 64, 64), pl.ds(c * DH, DH)])
        _emit(6)
        pltpu.sync_copy(
            wbuf.at[pl.ds(0, 16)],
            out_hbm.at[pl.ds(s * NPT + 6 * 64, 16), pl.ds(c * DH, DH)])


def kernel(x, edge_index, edge_weight, W, b, gamma, beta, edge_enc):
    ei = edge_index.astype(jnp.int32)
    scale = gamma * (1.0 / jnp.sqrt(jnp.float32(1.0 + 1e-5)))
    w_eff = (W * scale[:, None]).T.reshape(D, 2, DH).swapaxes(0, 1)
    w_eff = w_eff.reshape(2 * D, DH)
    b_eff = jnp.tile((b * scale + beta).reshape(2, 1, DH), (1, 8, 1))
    b_eff = b_eff.reshape(16, DH)

    npad = EF - E - N
    loop = jnp.arange(N, dtype=jnp.int32)
    row_full = jnp.concatenate(
        [ei[0], loop, jnp.zeros((npad,), jnp.int32)])
    col_full = jnp.concatenate(
        [ei[1], loop, jnp.full((npad,), PAD_NODE, jnp.int32)])
    ew_full = jnp.concatenate(
        [edge_weight * edge_enc.reshape(()), jnp.ones((N,), jnp.float32),
         jnp.zeros((npad,), jnp.float32)])

    deg = _deg_kernel(col_full)
    deg2d = deg[:, :N].T
    hcat = _linear(x, w_eff, b_eff, deg2d)
    out = _agg_kernel(row_full, col_full, ew_full, hcat, deg)
    return out


# speedup vs baseline: 9.8457x; 1.0557x over previous
import functools

import jax
import jax.numpy as jnp
from jax import lax
from jax.experimental import pallas as pl
from jax.experimental.pallas import tpu as pltpu
from jax.experimental.pallas import tpu_sc as plsc

N = 10000
NP = 10240
D = 256
DH = 128
E = 160000
EF = 172032
PAD_NODE = 10200
L = 16
NSUB = 16
NW = 32
EPT = EF // NSUB
G = 112
NB = EPT // G
EPTA = EF // NW
GA = 112
NBA = EPTA // GA
NPT = NP // NSUB

_mesh = plsc.VectorSubcoreMesh(core_axis_name="c", subcore_axis_name="s")


def _mm_body(x_ref, w_ref, b_ref, d_ref, o_ref):
    h = jnp.dot(x_ref[...], w_ref[...], preferred_element_type=jnp.float32)
    deg = d_ref[:, 0:1] + d_ref[:, 1:2]
    o_ref[...] = (h + b_ref[0:1, :]) * lax.rsqrt(deg)


def _linear(x, w_eff, b_eff, deg2d):
    bm = 1000
    return pl.pallas_call(
        _mm_body,
        grid=(2, N // bm),
        in_specs=[
            pl.BlockSpec((bm, D), lambda j, i: (i, 0)),
            pl.BlockSpec((D, DH), lambda j, i: (j, 0)),
            pl.BlockSpec((8, DH), lambda j, i: (j, 0)),
            pl.BlockSpec((bm, 2), lambda j, i: (i, 0)),
        ],
        out_specs=pl.BlockSpec((bm, DH), lambda j, i: (j * (N // bm) + i, 0)),
        out_shape=jax.ShapeDtypeStruct((2 * N, DH), jnp.float32),
    )(x, w_eff, b_eff, deg2d)


@functools.partial(
    pl.kernel,
    out_type=jax.ShapeDtypeStruct((2, NP), jnp.float32),
    mesh=_mesh,
    scratch_types=[
        pltpu.VMEM((GA,), jnp.int32),
        pltpu.VMEM((GA,), jnp.int32),
        pltpu.VMEM((GA, DH), jnp.float32),
        pltpu.VMEM((64, DH), jnp.float32),
        pltpu.VMEM((NPT,), jnp.float32),
        pltpu.VMEM_SHARED((NP, DH), jnp.float32),
        pltpu.SemaphoreType.DMA,
        pltpu.SemaphoreType.DMA,
    ],
)
def _deg_kernel(col_hbm, deg_hbm, cidx0, cidx1, ones, wbuf, dbuf, hist,
                sem0, sem1):
    c = lax.axis_index("c")
    s = lax.axis_index("s")
    z16 = jnp.zeros((L,), jnp.float32)
    o16 = jnp.ones((L,), jnp.float32)
    it16 = lax.iota(jnp.int32, L)

    def _zw(r, _):
        for j in range(DH // L):
            wbuf[r, pl.ds(j * L, L)] = z16
        return 0
    lax.fori_loop(0, 64, _zw, 0)
    for t in range(NPT // 64):
        pltpu.sync_copy(wbuf, hist.at[pl.ds(s * NPT + t * 64, 64)])

    def _o(i, _):
        for j in range(DH // L):
            ones[i, pl.ds(j * L, L)] = o16
        return 0
    lax.fori_loop(0, GA, _o, 0)

    plsc.subcore_barrier()

    ebase = (c * NSUB + s) * EPTA

    def _ld(k, cref, sem):
        return pltpu.async_copy(col_hbm.at[pl.ds(ebase + k * GA, GA)],
                                cref, sem)

    _ld(0, cidx0, sem0)

    def _pair(p, _):
        _ld(2 * p + 1, cidx1, sem1)
        pltpu.make_async_copy(col_hbm.at[pl.ds(0, GA)], cidx0, sem0).wait()
        pltpu.sync_copy(ones, hist.at[cidx0], add=True)

        @pl.when(2 * p + 2 < NBA)
        def _():
            _ld(2 * p + 2, cidx0, sem0)
        pltpu.make_async_copy(col_hbm.at[pl.ds(0, GA)], cidx1, sem1).wait()
        pltpu.sync_copy(ones, hist.at[cidx1], add=True)
        return 0
    lax.fori_loop(0, NBA // 2, _pair, 0)

    plsc.subcore_barrier()

    for t in range(NPT // 64):
        pltpu.sync_copy(hist.at[pl.ds(s * NPT + t * 64, 64)], wbuf)

        def _grp(g, _, t=t):
            v = z16
            for l in range(L):
                x = wbuf[g * L + l, pl.ds(0, L)][0]
                v = jnp.where(it16 == l,
                              jnp.full((L,), x, jnp.float32), v)
            dbuf[pl.ds(t * 64 + g * L, L)] = v
            return 0
        lax.fori_loop(0, 64 // L, _grp, 0)
    pltpu.sync_copy(dbuf, deg_hbm.at[c, pl.ds(s * NPT, NPT)])


def _rsqrt16(d):
    y = jnp.full((L,), 2.0 ** -10, jnp.float32)
    for thresh, seed in ((65536.0, 2.0 ** -8), (4096.0, 2.0 ** -6),
                         (256.0, 2.0 ** -4), (16.0, 2.0 ** -2)):
        y = jnp.where(d < thresh, jnp.full((L,), seed, jnp.float32), y)
    for _ in range(8):
        y = y * (1.5 - 0.5 * d * y * y)
    return y


@functools.partial(
    pl.kernel,
    out_type=jax.ShapeDtypeStruct((N, D), jnp.float32),
    mesh=_mesh,
    scratch_types=[
        [pltpu.VMEM((G,), jnp.int32) for _ in range(2)],
        [pltpu.VMEM((G,), jnp.int32) for _ in range(2)],
        [pltpu.VMEM((G + L,), jnp.float32) for _ in range(2)],
        [pltpu.VMEM((G, DH), jnp.float32) for _ in range(2)],
        pltpu.VMEM((64, DH), jnp.float32),
        pltpu.VMEM((NPT + L,), jnp.float32),
        pltpu.VMEM((NPT,), jnp.float32),
        pltpu.VMEM_SHARED((NP, DH), jnp.float32),
        [pltpu.SemaphoreType.DMA for _ in range(2)],
        [pltpu.SemaphoreType.DMA for _ in range(2)],
    ],
)
def _agg_kernel(row_hbm, col_hbm, ew_hbm, hcat_hbm, deg_hbm, out_hbm,
                ridx, cidx, ewb, gbuf, wbuf, disv, dtmp, acc, sem, isem):
    c = lax.axis_index("c")
    s = lax.axis_index("s")
    z16 = jnp.zeros((L,), jnp.float32)

    def _zw(r, _):
        for j in range(DH // L):
            wbuf[r, pl.ds(j * L, L)] = z16
        return 0
    lax.fori_loop(0, 64, _zw, 0)
    for t in range(NPT // 64):
        pltpu.sync_copy(wbuf, acc.at[pl.ds(s * NPT + t * 64, 64)])

    plsc.subcore_barrier()

    ebase = s * EPT

    def _prep(g, q):
        b = ebase + g * G
        br = c * EF + b
        pltpu.async_copy(row_hbm.at[pl.ds(br, G)], ridx[q], isem[q])
        pltpu.async_copy(col_hbm.at[pl.ds(b, G)], cidx[q], isem[q])
        pltpu.async_copy(ew_hbm.at[pl.ds(b, G)], ewb[q].at[pl.ds(0, G)],
                         isem[q])
        pltpu.make_async_copy(row_hbm.at[pl.ds(br, G)], ridx[q],
                              isem[q]).wait()
        pltpu.make_async_copy(col_hbm.at[pl.ds(b, G)], cidx[q],
                              isem[q]).wait()
        pltpu.make_async_copy(ew_hbm.at[pl.ds(b, G)], ewb[q].at[pl.ds(0, G)],
                              isem[q]).wait()
        pltpu.async_copy(hcat_hbm.at[ridx[q]], gbuf[q], sem[q])

    def _proc(q):
        pltpu.make_async_copy(hcat_hbm.at[ridx[q]], gbuf[q], sem[q]).wait()

        def _edge(i, _):
            cf = jnp.full((L,), ewb[q][pl.ds(i, L)][0], jnp.float32)
            for j in range(DH // L):
                sl = pl.ds(j * L, L)
                gbuf[q][i, sl] = gbuf[q][i, sl] * cf
            return 0
        lax.fori_loop(0, G, _edge, 0)
        pltpu.sync_copy(gbuf[q], acc.at[cidx[q]], add=True)

    _prep(0, 0)

    def _pair(p, _):
        _prep(2 * p + 1, 1)
        _proc(0)

        @pl.when(2 * p + 2 < NB)
        def _():
            _prep(2 * p + 2, 0)
        _proc(1)
        return 0
    lax.fori_loop(0, NB // 2, _pair, 0)

    plsc.subcore_barrier()

    pltpu.sync_copy(deg_hbm.at[0, pl.ds(s * NPT, NPT)],
                    disv.at[pl.ds(0, NPT)])
    pltpu.sync_copy(deg_hbm.at[1, pl.ds(s * NPT, NPT)], dtmp)

    def _dis(k, _):
        sl = pl.ds(k * L, L)
        disv[sl] = _rsqrt16(disv[sl] + dtmp[sl])
        return 0
    lax.fori_loop(0, NPT // L, _dis, 0)

    def _emit(t):
        rbase = s * NPT + t * 64
        pltpu.sync_copy(acc.at[pl.ds(rbase, 64)], wbuf)

        def _row(r, _):
            dv = jnp.full((L,), disv[pl.ds(t * 64 + r, L)][0], jnp.float32)
            for j in range(DH // L):
                sl = pl.ds(j * L, L)
                wbuf[r, sl] = wbuf[r, sl] * dv
            return 0
        lax.fori_loop(0, 64, _row, 0)

    @pl.when(s < NSUB - 1)
    def _full():
        for t in range(NPT // 64):
            _emit(t)
            pltpu.sync_copy(
                wbuf,
                out_hbm.at[pl.ds(s * NPT + t * 64, 64), pl.ds(c * DH, DH)])

    @pl.when(s == NSUB - 1)
    def _last():
        for t in range(6):
            _emit(t)
            pltpu.sync_copy(
                wbuf,
                out_hbm.at[pl.ds(s * NPT + t * 64, 64), pl.ds(c * DH, DH)])
        _emit(6)
        pltpu.sync_copy(
            wbuf.at[pl.ds(0, 16)],
            out_hbm.at[pl.ds(s * NPT + 6 * 64, 16), pl.ds(c * DH, DH)])


def kernel(x, edge_index, edge_weight, W, b, gamma, beta, edge_enc):
    ei = edge_index.astype(jnp.int32)
    scale = gamma * (1.0 / jnp.sqrt(jnp.float32(1.0 + 1e-5)))
    w_eff = (W * scale[:, None]).T.reshape(D, 2, DH).swapaxes(0, 1)
    w_eff = w_eff.reshape(2 * D, DH)
    b_eff = jnp.tile((b * scale + beta).reshape(2, 1, DH), (1, 8, 1))
    b_eff = b_eff.reshape(16, DH)

    npad = EF - E - N
    loop = jnp.arange(N, dtype=jnp.int32)
    row_full = jnp.concatenate(
        [ei[0], loop, jnp.zeros((npad,), jnp.int32)])
    rowcat = jnp.concatenate([row_full, row_full + N])
    col_full = jnp.concatenate(
        [ei[1], loop, jnp.full((npad,), PAD_NODE, jnp.int32)])
    ew_full = jnp.concatenate(
        [edge_weight * edge_enc.reshape(()), jnp.ones((N,), jnp.float32),
         jnp.zeros((npad,), jnp.float32)])

    deg = _deg_kernel(col_full)
    deg2d = deg[:, :N].T
    hcat = _linear(x, w_eff, b_eff, deg2d)
    out = _agg_kernel(rowcat, col_full, ew_full, hcat, deg)
    return out


# speedup vs baseline: 9.8468x; 1.0001x over previous
import functools

import jax
import jax.numpy as jnp
from jax import lax
from jax.experimental import pallas as pl
from jax.experimental.pallas import tpu as pltpu
from jax.experimental.pallas import tpu_sc as plsc

N = 10000
NP = 10240
D = 256
DH = 128
E = 160000
EF = 172032
PAD_NODE = 10200
L = 16
NSUB = 16
NW = 32
EPT = EF // NSUB
G = 112
NB = EPT // G
EPTA = EF // NW
GA = 112
NBA = EPTA // GA
NPT = NP // NSUB

_mesh = plsc.VectorSubcoreMesh(core_axis_name="c", subcore_axis_name="s")


def _mm_body(x_ref, w_ref, b_ref, d_ref, o_ref):
    h = jnp.dot(x_ref[...], w_ref[...], preferred_element_type=jnp.float32)
    deg = d_ref[:, 0:1] + d_ref[:, 1:2]
    o_ref[...] = (h + b_ref[0:1, :]) * lax.rsqrt(deg)


def _linear(x, w_eff, b_eff, deg2d):
    bm = 1000
    return pl.pallas_call(
        _mm_body,
        grid=(2, N // bm),
        in_specs=[
            pl.BlockSpec((bm, D), lambda j, i: (i, 0)),
            pl.BlockSpec((D, DH), lambda j, i: (j, 0)),
            pl.BlockSpec((8, DH), lambda j, i: (j, 0)),
            pl.BlockSpec((bm, 2), lambda j, i: (i, 0)),
        ],
        out_specs=pl.BlockSpec((bm, DH), lambda j, i: (j * (N // bm) + i, 0)),
        out_shape=jax.ShapeDtypeStruct((2 * N, DH), jnp.float32),
    )(x, w_eff, b_eff, deg2d)


@functools.partial(
    pl.kernel,
    out_type=jax.ShapeDtypeStruct((2, NP), jnp.float32),
    mesh=_mesh,
    scratch_types=[
        pltpu.VMEM((GA,), jnp.int32),
        pltpu.VMEM((GA,), jnp.int32),
        pltpu.VMEM((GA, DH), jnp.float32),
        pltpu.VMEM((64, DH), jnp.float32),
        pltpu.VMEM((NPT,), jnp.float32),
        pltpu.VMEM_SHARED((NP, DH), jnp.float32),
        pltpu.SemaphoreType.DMA,
        pltpu.SemaphoreType.DMA,
    ],
)
def _deg_kernel(col_hbm, deg_hbm, cidx0, cidx1, ones, wbuf, dbuf, hist,
                sem0, sem1):
    c = lax.axis_index("c")
    s = lax.axis_index("s")
    z16 = jnp.zeros((L,), jnp.float32)
    o16 = jnp.ones((L,), jnp.float32)
    it16 = lax.iota(jnp.int32, L)

    def _zw(r, _):
        for j in range(DH // L):
            wbuf[r, pl.ds(j * L, L)] = z16
        return 0
    lax.fori_loop(0, 64, _zw, 0)
    for t in range(NPT // 64):
        pltpu.sync_copy(wbuf, hist.at[pl.ds(s * NPT + t * 64, 64)])

    def _o(i, _):
        for j in range(DH // L):
            ones[i, pl.ds(j * L, L)] = o16
        return 0
    lax.fori_loop(0, GA, _o, 0)

    plsc.subcore_barrier()

    ebase = (c * NSUB + s) * EPTA

    def _ld(k, cref, sem):
        return pltpu.async_copy(col_hbm.at[pl.ds(ebase + k * GA, GA)],
                                cref, sem)

    _ld(0, cidx0, sem0)

    def _pair(p, _):
        _ld(2 * p + 1, cidx1, sem1)
        pltpu.make_async_copy(col_hbm.at[pl.ds(0, GA)], cidx0, sem0).wait()
        pltpu.sync_copy(ones, hist.at[cidx0], add=True)

        @pl.when(2 * p + 2 < NBA)
        def _():
            _ld(2 * p + 2, cidx0, sem0)
        pltpu.make_async_copy(col_hbm.at[pl.ds(0, GA)], cidx1, sem1).wait()
        pltpu.sync_copy(ones, hist.at[cidx1], add=True)
        return 0
    lax.fori_loop(0, NBA // 2, _pair, 0)

    plsc.subcore_barrier()

    for t in range(NPT // 64):
        pltpu.sync_copy(hist.at[pl.ds(s * NPT + t * 64, 64)], wbuf)

        def _grp(g, _, t=t):
            v = z16
            for l in range(L):
                x = wbuf[g * L + l, pl.ds(0, L)][0]
                v = jnp.where(it16 == l,
                              jnp.full((L,), x, jnp.float32), v)
            dbuf[pl.ds(t * 64 + g * L, L)] = v
            return 0
        lax.fori_loop(0, 64 // L, _grp, 0)
    pltpu.sync_copy(dbuf, deg_hbm.at[c, pl.ds(s * NPT, NPT)])


def _rsqrt16(d):
    y = jnp.full((L,), 2.0 ** -10, jnp.float32)
    for thresh, seed in ((65536.0, 2.0 ** -8), (4096.0, 2.0 ** -6),
                         (256.0, 2.0 ** -4), (16.0, 2.0 ** -2)):
        y = jnp.where(d < thresh, jnp.full((L,), seed, jnp.float32), y)
    for _ in range(8):
        y = y * (1.5 - 0.5 * d * y * y)
    return y


@functools.partial(
    pl.kernel,
    out_type=jax.ShapeDtypeStruct((N, D), jnp.float32),
    mesh=_mesh,
    scratch_types=[
        [pltpu.VMEM((G,), jnp.int32) for _ in range(2)],
        [pltpu.VMEM((G,), jnp.int32) for _ in range(2)],
        [pltpu.VMEM((G + L,), jnp.float32) for _ in range(2)],
        [pltpu.VMEM((G, DH), jnp.float32) for _ in range(2)],
        pltpu.VMEM((64, DH), jnp.float32),
        pltpu.VMEM((NPT + L,), jnp.float32),
        pltpu.VMEM((NPT,), jnp.float32),
        pltpu.VMEM_SHARED((NP, DH), jnp.float32),
        [pltpu.SemaphoreType.DMA for _ in range(2)],
        [pltpu.SemaphoreType.DMA for _ in range(2)],
        [pltpu.SemaphoreType.DMA for _ in range(2)],
    ],
)
def _agg_kernel(row_hbm, col_hbm, ew_hbm, hcat_hbm, deg_hbm, out_hbm,
                ridx, cidx, ewb, gbuf, wbuf, disv, dtmp, acc, sem, isem,
                ssem):
    c = lax.axis_index("c")
    s = lax.axis_index("s")
    z16 = jnp.zeros((L,), jnp.float32)

    def _zw(r, _):
        for j in range(DH // L):
            wbuf[r, pl.ds(j * L, L)] = z16
        return 0
    lax.fori_loop(0, 64, _zw, 0)
    for t in range(NPT // 64):
        pltpu.sync_copy(wbuf, acc.at[pl.ds(s * NPT + t * 64, 64)])

    plsc.subcore_barrier()

    ebase = s * EPT

    def _prep(g, q):
        @pl.when(g >= 2)
        def _():
            pltpu.make_async_copy(gbuf[q], acc.at[cidx[q]], ssem[q]).wait()
        b = ebase + g * G
        br = c * EF + b
        pltpu.async_copy(row_hbm.at[pl.ds(br, G)], ridx[q], isem[q])
        pltpu.async_copy(col_hbm.at[pl.ds(b, G)], cidx[q], isem[q])
        pltpu.async_copy(ew_hbm.at[pl.ds(b, G)], ewb[q].at[pl.ds(0, G)],
                         isem[q])
        pltpu.make_async_copy(row_hbm.at[pl.ds(br, G)], ridx[q],
                              isem[q]).wait()
        pltpu.make_async_copy(col_hbm.at[pl.ds(b, G)], cidx[q],
                              isem[q]).wait()
        pltpu.make_async_copy(ew_hbm.at[pl.ds(b, G)], ewb[q].at[pl.ds(0, G)],
                              isem[q]).wait()
        pltpu.async_copy(hcat_hbm.at[ridx[q]], gbuf[q], sem[q])

    def _proc(q):
        pltpu.make_async_copy(hcat_hbm.at[ridx[q]], gbuf[q], sem[q]).wait()

        def _edge(i, _):
            cf = jnp.full((L,), ewb[q][pl.ds(i, L)][0], jnp.float32)
            for j in range(DH // L):
                sl = pl.ds(j * L, L)
                gbuf[q][i, sl] = gbuf[q][i, sl] * cf
            return 0
        lax.fori_loop(0, G, _edge, 0)
        pltpu.async_copy(gbuf[q], acc.at[cidx[q]], ssem[q], add=True)

    _prep(0, 0)

    def _pair(p, _):
        _prep(2 * p + 1, 1)
        _proc(0)

        @pl.when(2 * p + 2 < NB)
        def _():
            _prep(2 * p + 2, 0)
        _proc(1)
        return 0
    lax.fori_loop(0, NB // 2, _pair, 0)
    pltpu.make_async_copy(gbuf[0], acc.at[cidx[0]], ssem[0]).wait()
    pltpu.make_async_copy(gbuf[1], acc.at[cidx[1]], ssem[1]).wait()

    plsc.subcore_barrier()

    pltpu.sync_copy(deg_hbm.at[0, pl.ds(s * NPT, NPT)],
                    disv.at[pl.ds(0, NPT)])
    pltpu.sync_copy(deg_hbm.at[1, pl.ds(s * NPT, NPT)], dtmp)

    def _dis(k, _):
        sl = pl.ds(k * L, L)
        disv[sl] = _rsqrt16(disv[sl] + dtmp[sl])
        return 0
    lax.fori_loop(0, NPT // L, _dis, 0)

    def _emit(t):
        rbase = s * NPT + t * 64
        pltpu.sync_copy(acc.at[pl.ds(rbase, 64)], wbuf)

        def _row(r, _):
            dv = jnp.full((L,), disv[pl.ds(t * 64 + r, L)][0], jnp.float32)
            for j in range(DH // L):
                sl = pl.ds(j * L, L)
                wbuf[r, sl] = wbuf[r, sl] * dv
            return 0
        lax.fori_loop(0, 64, _row, 0)

    @pl.when(s < NSUB - 1)
    def _full():
        for t in range(NPT // 64):
            _emit(t)
            pltpu.sync_copy(
                wbuf,
                out_hbm.at[pl.ds(s * NPT + t * 64, 64), pl.ds(c * DH, DH)])

    @pl.when(s == NSUB - 1)
    def _last():
        for t in range(6):
            _emit(t)
            pltpu.sync_copy(
                wbuf,
                out_hbm.at[pl.ds(s * NPT + t * 64, 64), pl.ds(c * DH, DH)])
        _emit(6)
        pltpu.sync_copy(
            wbuf.at[pl.ds(0, 16)],
            out_hbm.at[pl.ds(s * NPT + 6 * 64, 16), pl.ds(c * DH, DH)])


def kernel(x, edge_index, edge_weight, W, b, gamma, beta, edge_enc):
    ei = edge_index.astype(jnp.int32)
    scale = gamma * (1.0 / jnp.sqrt(jnp.float32(1.0 + 1e-5)))
    w_eff = (W * scale[:, None]).T.reshape(D, 2, DH).swapaxes(0, 1)
    w_eff = w_eff.reshape(2 * D, DH)
    b_eff = jnp.tile((b * scale + beta).reshape(2, 1, DH), (1, 8, 1))
    b_eff = b_eff.reshape(16, DH)

    npad = EF - E - N
    loop = jnp.arange(N, dtype=jnp.int32)
    row_full = jnp.concatenate(
        [ei[0], loop, jnp.zeros((npad,), jnp.int32)])
    rowcat = jnp.concatenate([row_full, row_full + N])
    col_full = jnp.concatenate(
        [ei[1], loop, jnp.full((npad,), PAD_NODE, jnp.int32)])
    ew_full = jnp.concatenate(
        [edge_weight * edge_enc.reshape(()), jnp.ones((N,), jnp.float32),
         jnp.zeros((npad,), jnp.float32)])

    deg = _deg_kernel(col_full)
    deg2d = deg[:, :N].T
    hcat = _linear(x, w_eff, b_eff, deg2d)
    out = _agg_kernel(rowcat, col_full, ew_full, hcat, deg)
    return out
